# baseline (speedup 1.0000x reference)
import functools

import jax
import jax.numpy as jnp
from jax import lax
from jax.experimental import pallas as pl
from jax.experimental.pallas import tpu as pltpu
from jax.experimental.pallas import tpu_sc as plsc

B, S, F = 4, 4096, 2048
CTX = 2048
ROW0 = S - CTX
NC, NS, L = 2, 16, 16
NW = NC * NS
CPW = (B * F) // NW
WPB = F // CPW
R = 128
NCHUNK = CTX // R
G = 4
NG = CPW // (G * L)

_MESH = plsc.VectorSubcoreMesh(
    core_axis_name="c", subcore_axis_name="s", num_cores=NC, num_subcores=NS
)


@functools.partial(
    pl.kernel,
    out_type=(
        jax.ShapeDtypeStruct((B, 1, F), jnp.float32),
        jax.ShapeDtypeStruct((B, 1, F), jnp.float32),
    ),
    mesh=_MESH,
    scratch_types=[
        pltpu.VMEM((R, CPW), jnp.float32),
        pltpu.VMEM((R, CPW), jnp.float32),
        pltpu.VMEM((CPW,), jnp.float32),
        pltpu.VMEM((CPW,), jnp.float32),
        pltpu.VMEM((CPW,), jnp.float32),
        pltpu.VMEM((CPW,), jnp.float32),
        pltpu.VMEM((CPW,), jnp.float32),
    ],
)
def _sum_extraction(data_hbm, mask_hbm, pooled_hbm, mmax_hbm,
                    dbuf, mbuf, acc_dm, acc_m, acc_d, acc_mx, obuf):
    wid = lax.axis_index("s") * NC + lax.axis_index("c")
    b = wid // WPB
    c0 = (wid % WPB) * CPW

    zeros = jnp.zeros((L,), jnp.float32)
    ninf = jnp.full((L,), -jnp.inf, jnp.float32)
    for j in range(CPW // L):
        sl = pl.ds(j * L, L)
        acc_dm[sl] = zeros
        acc_m[sl] = zeros
        acc_d[sl] = zeros
        acc_mx[sl] = ninf

    def chunk_body(ck, carry):
        row0 = ROW0 + ck * R
        pltpu.sync_copy(data_hbm.at[b, pl.ds(row0, R), pl.ds(c0, CPW)], dbuf)
        pltpu.sync_copy(mask_hbm.at[b, pl.ds(row0, R), pl.ds(c0, CPW)], mbuf)
        for g in range(NG):
            base = g * G * L

            def row_body(r, acc):
                new_dm, new_m, new_d, new_mx = [], [], [], []
                for j in range(G):
                    sl = pl.ds(base + j * L, L)
                    d = dbuf[r, sl]
                    m = mbuf[r, sl]
                    new_dm.append(acc[j] + d * m)
                    new_m.append(acc[G + j] + m)
                    new_d.append(acc[2 * G + j] + d)
                    new_mx.append(jnp.maximum(acc[3 * G + j], m))
                return tuple(new_dm + new_m + new_d + new_mx)

            init = tuple([zeros] * (3 * G) + [ninf] * G)
            res = lax.fori_loop(0, R, row_body, init)
            for j in range(G):
                sl = pl.ds(base + j * L, L)
                acc_dm[sl] = acc_dm[sl] + res[j]
                acc_m[sl] = acc_m[sl] + res[G + j]
                acc_d[sl] = acc_d[sl] + res[2 * G + j]
                acc_mx[sl] = jnp.maximum(acc_mx[sl], res[3 * G + j])
        return carry

    lax.fori_loop(0, NCHUNK, chunk_body, 0)

    inv_n = jnp.float32(1.0 / CTX)
    eps = jnp.float32(1e-8)
    for j in range(CPW // L):
        sl = pl.ds(j * L, L)
        msum = acc_m[sl]
        obuf[sl] = jnp.where(
            msum == 0.0, acc_d[sl] * inv_n, acc_dm[sl] / (msum + eps)
        )
    pltpu.sync_copy(obuf, pooled_hbm.at[b, 0, pl.ds(c0, CPW)])
    for j in range(CPW // L):
        sl = pl.ds(j * L, L)
        obuf[sl] = acc_mx[sl]
    pltpu.sync_copy(obuf, mmax_hbm.at[b, 0, pl.ds(c0, CPW)])


def kernel(data, mask):
    return _sum_extraction(data, mask)


# speedup vs baseline: 1.5099x; 1.5099x over previous
import functools

import jax
import jax.numpy as jnp
from jax import lax
from jax.experimental import pallas as pl
from jax.experimental.pallas import tpu as pltpu
from jax.experimental.pallas import tpu_sc as plsc

B, S, F = 4, 4096, 2048
CTX = 2048
ROW0 = S - CTX
NC, NS, L = 2, 16, 16
NW = NC * NS
CPW = (B * F) // NW
WPB = F // CPW
R = 64
NCHUNK = CTX // R
G = 4
NG = CPW // (G * L)

_MESH = plsc.VectorSubcoreMesh(
    core_axis_name="c", subcore_axis_name="s", num_cores=NC, num_subcores=NS
)


@functools.partial(
    pl.kernel,
    out_type=(
        jax.ShapeDtypeStruct((B, 1, F), jnp.float32),
        jax.ShapeDtypeStruct((B, 1, F), jnp.float32),
    ),
    mesh=_MESH,
    scratch_types=[
        pltpu.VMEM((2, R, CPW), jnp.float32),
        pltpu.VMEM((2, R, CPW), jnp.float32),
        pltpu.VMEM((CPW,), jnp.float32),
        pltpu.VMEM((CPW,), jnp.float32),
        pltpu.VMEM((CPW,), jnp.float32),
        pltpu.VMEM((CPW,), jnp.float32),
        pltpu.VMEM((CPW,), jnp.float32),
        pltpu.SemaphoreType.DMA,
        pltpu.SemaphoreType.DMA,
        pltpu.SemaphoreType.DMA,
        pltpu.SemaphoreType.DMA,
    ],
)
def _sum_extraction(data_hbm, mask_hbm, pooled_hbm, mmax_hbm,
                    dbuf, mbuf, acc_dm, acc_m, acc_d, acc_mx, obuf,
                    sd0, sd1, sm0, sm1):
    wid = lax.axis_index("s") * NC + lax.axis_index("c")
    b = wid // WPB
    c0 = (wid % WPB) * CPW
    sems_d = (sd0, sd1)
    sems_m = (sm0, sm1)

    def src_d(ck):
        return data_hbm.at[b, pl.ds(ROW0 + ck * R, R), pl.ds(c0, CPW)]

    def src_m(ck):
        return mask_hbm.at[b, pl.ds(ROW0 + ck * R, R), pl.ds(c0, CPW)]

    def start(ck, buf):
        pltpu.async_copy(src_d(ck), dbuf.at[buf], sems_d[buf])
        pltpu.async_copy(src_m(ck), mbuf.at[buf], sems_m[buf])

    def wait(buf):
        pltpu.make_async_copy(src_d(0), dbuf.at[buf], sems_d[buf]).wait()
        pltpu.make_async_copy(src_m(0), mbuf.at[buf], sems_m[buf]).wait()

    zeros = jnp.zeros((L,), jnp.float32)
    ninf = jnp.full((L,), -jnp.inf, jnp.float32)
    for j in range(CPW // L):
        sl = pl.ds(j * L, L)
        acc_dm[sl] = zeros
        acc_m[sl] = zeros
        acc_d[sl] = zeros
        acc_mx[sl] = ninf

    def compute(buf):
        for g in range(NG):
            base = g * G * L

            def row_body(r, acc):
                new_dm, new_m, new_d, new_mx = [], [], [], []
                for j in range(G):
                    sl = pl.ds(base + j * L, L)
                    d = dbuf[buf, r, sl]
                    m = mbuf[buf, r, sl]
                    new_dm.append(acc[j] + d * m)
                    new_m.append(acc[G + j] + m)
                    new_d.append(acc[2 * G + j] + d)
                    new_mx.append(jnp.maximum(acc[3 * G + j], m))
                return tuple(new_dm + new_m + new_d + new_mx)

            init = tuple([zeros] * (3 * G) + [ninf] * G)
            res = lax.fori_loop(0, R, row_body, init)
            for j in range(G):
                sl = pl.ds(base + j * L, L)
                acc_dm[sl] = acc_dm[sl] + res[j]
                acc_m[sl] = acc_m[sl] + res[G + j]
                acc_d[sl] = acc_d[sl] + res[2 * G + j]
                acc_mx[sl] = jnp.maximum(acc_mx[sl], res[3 * G + j])

    start(0, 0)
    start(1, 1)

    def pair_body(cp, carry):
        g0 = 2 * cp
        wait(0)
        compute(0)
        start(g0 + 2, 0)
        wait(1)
        compute(1)
        start(g0 + 3, 1)
        return carry

    lax.fori_loop(0, NCHUNK // 2 - 1, pair_body, 0)
    wait(0)
    compute(0)
    wait(1)
    compute(1)

    inv_n = jnp.float32(1.0 / CTX)
    eps = jnp.float32(1e-8)
    for j in range(CPW // L):
        sl = pl.ds(j * L, L)
        msum = acc_m[sl]
        obuf[sl] = jnp.where(
            msum == 0.0, acc_d[sl] * inv_n, acc_dm[sl] / (msum + eps)
        )
    pltpu.sync_copy(obuf, pooled_hbm.at[b, 0, pl.ds(c0, CPW)])
    for j in range(CPW // L):
        sl = pl.ds(j * L, L)
        obuf[sl] = acc_mx[sl]
    pltpu.sync_copy(obuf, mmax_hbm.at[b, 0, pl.ds(c0, CPW)])


def kernel(data, mask):
    return _sum_extraction(data, mask)


# speedup vs baseline: 1.6420x; 1.0875x over previous
import functools

import jax
import jax.numpy as jnp
from jax import lax
from jax.experimental import pallas as pl
from jax.experimental.pallas import tpu as pltpu
from jax.experimental.pallas import tpu_sc as plsc

B, S, F = 4, 4096, 2048
CTX = 2048
ROW0 = S - CTX
NC, NS, L = 2, 16, 16
NW = NC * NS
CPW = (B * F) // NW
WPB = F // CPW
R = 64
NCHUNK = CTX // R
G = 4
NG = CPW // (G * L)

_MESH = plsc.VectorSubcoreMesh(
    core_axis_name="c", subcore_axis_name="s", num_cores=NC, num_subcores=NS
)


@functools.partial(
    pl.kernel,
    out_type=(
        jax.ShapeDtypeStruct((B, 1, F), jnp.float32),
        jax.ShapeDtypeStruct((B, 1, F), jnp.float32),
    ),
    mesh=_MESH,
    scratch_types=[
        pltpu.VMEM((2, R, CPW), jnp.float32),
        pltpu.VMEM((2, R, CPW), jnp.float32),
        pltpu.VMEM((CPW,), jnp.float32),
        pltpu.VMEM((CPW,), jnp.float32),
        pltpu.VMEM((CPW,), jnp.float32),
        pltpu.VMEM((CPW,), jnp.float32),
        pltpu.VMEM((CPW,), jnp.float32),
        pltpu.SemaphoreType.DMA,
        pltpu.SemaphoreType.DMA,
        pltpu.SemaphoreType.DMA,
        pltpu.SemaphoreType.DMA,
    ],
)
def _sum_extraction(data_hbm, mask_hbm, pooled_hbm, mmax_hbm,
                    dbuf, mbuf, acc_dm, acc_m, acc_d, acc_mx, obuf,
                    sd0, sd1, sm0, sm1):
    wid = lax.axis_index("s") * NC + lax.axis_index("c")
    b = wid // WPB
    c0 = (wid % WPB) * CPW
    sems_d = (sd0, sd1)
    sems_m = (sm0, sm1)

    def src_d(ck):
        return data_hbm.at[b, pl.ds(ROW0 + ck * R, R), pl.ds(c0, CPW)]

    def src_m(ck):
        return mask_hbm.at[b, pl.ds(ROW0 + ck * R, R), pl.ds(c0, CPW)]

    def start(ck, buf):
        pltpu.async_copy(src_d(ck), dbuf.at[buf], sems_d[buf])
        pltpu.async_copy(src_m(ck), mbuf.at[buf], sems_m[buf])

    def wait(buf):
        pltpu.make_async_copy(src_d(0), dbuf.at[buf], sems_d[buf]).wait()
        pltpu.make_async_copy(src_m(0), mbuf.at[buf], sems_m[buf]).wait()

    zeros = jnp.zeros((L,), jnp.float32)
    ninf = jnp.full((L,), -jnp.inf, jnp.float32)
    for j in range(CPW // L):
        sl = pl.ds(j * L, L)
        acc_dm[sl] = zeros
        acc_m[sl] = zeros
        acc_d[sl] = zeros
        acc_mx[sl] = ninf

    def compute(buf):
        for g in range(NG):
            base = g * G * L
            init = tuple([zeros] * (3 * G) + [ninf] * G)

            @plsc.parallel_loop(0, R, unroll=8, carry=init)
            def res(r, acc):
                new_dm, new_m, new_d, new_mx = [], [], [], []
                for j in range(G):
                    sl = pl.ds(base + j * L, L)
                    d = dbuf[buf, r, sl]
                    m = mbuf[buf, r, sl]
                    new_dm.append(acc[j] + d * m)
                    new_m.append(acc[G + j] + m)
                    new_d.append(acc[2 * G + j] + d)
                    new_mx.append(jnp.maximum(acc[3 * G + j], m))
                return tuple(new_dm + new_m + new_d + new_mx)
            for j in range(G):
                sl = pl.ds(base + j * L, L)
                acc_dm[sl] = acc_dm[sl] + res[j]
                acc_m[sl] = acc_m[sl] + res[G + j]
                acc_d[sl] = acc_d[sl] + res[2 * G + j]
                acc_mx[sl] = jnp.maximum(acc_mx[sl], res[3 * G + j])

    start(0, 0)
    start(1, 1)

    def pair_body(cp, carry):
        g0 = 2 * cp
        wait(0)
        compute(0)
        start(g0 + 2, 0)
        wait(1)
        compute(1)
        start(g0 + 3, 1)
        return carry

    lax.fori_loop(0, NCHUNK // 2 - 1, pair_body, 0)
    wait(0)
    compute(0)
    wait(1)
    compute(1)

    inv_n = jnp.float32(1.0 / CTX)
    eps = jnp.float32(1e-8)
    for j in range(CPW // L):
        sl = pl.ds(j * L, L)
        msum = acc_m[sl]
        obuf[sl] = jnp.where(
            msum == 0.0, acc_d[sl] * inv_n, acc_dm[sl] / (msum + eps)
        )
    pltpu.sync_copy(obuf, pooled_hbm.at[b, 0, pl.ds(c0, CPW)])
    for j in range(CPW // L):
        sl = pl.ds(j * L, L)
        obuf[sl] = acc_mx[sl]
    pltpu.sync_copy(obuf, mmax_hbm.at[b, 0, pl.ds(c0, CPW)])


def kernel(data, mask):
    return _sum_extraction(data, mask)
